# baseline (speedup 1.0000x reference)
import functools

import jax
import jax.numpy as jnp
from jax import lax
from jax.experimental import pallas as pl
from jax.experimental.pallas import tpu as pltpu
from jax.experimental.pallas import tpu_sc as plsc

N = 10000
IN = 128
HID = 128
OUT = 128
H = 4
DH = 32
K = 16
P = 3
ORG = 25
SEV = 4

NPAD = 10240
NW = 32
CPW = NPAD // NW
B = 16
NBLK = CPW // B
ZW = HID + 16
BT = 512

_dot = functools.partial(jnp.dot, precision=lax.Precision.HIGHEST,
                         preferred_element_type=jnp.float32)


def _s1_body(x_ref, wp_ref, bp_ref, watt_ref, asrc_ref, adst_ref, wpat_ref,
             hp_ref, s_ref, ed_ref, hpat_ref):
    x = x_ref[...]
    h = jax.nn.gelu(_dot(x, wp_ref[...]) + bp_ref[...])
    hpat_ref[...] = _dot(h, wpat_ref[...])
    row = lax.broadcasted_iota(jnp.int32, (HID, H), 0) // DH
    col = lax.broadcasted_iota(jnp.int32, (HID, H), 1)
    M = (row == col).astype(jnp.float32)
    for p in range(P):
        hp = _dot(h, watt_ref[p])
        hp_ref[p] = hp
        s_ref[p] = _dot(hp * asrc_ref[p][None, :], M)
        ed_ref[p] = _dot(hp * adst_ref[p][None, :], M)


def _stage1(x_pad, W_proj, b_proj, watt, asrc, adst, W_pat):
    grid = (NPAD // BT,)
    return pl.pallas_call(
        _s1_body,
        grid=grid,
        in_specs=[
            pl.BlockSpec((BT, IN), lambda i: (i, 0)),
            pl.BlockSpec((IN, HID), lambda i: (0, 0)),
            pl.BlockSpec((1, HID), lambda i: (0, 0)),
            pl.BlockSpec((P, HID, HID), lambda i: (0, 0, 0)),
            pl.BlockSpec((P, HID), lambda i: (0, 0)),
            pl.BlockSpec((P, HID), lambda i: (0, 0)),
            pl.BlockSpec((HID, HID), lambda i: (0, 0)),
        ],
        out_specs=[
            pl.BlockSpec((P, BT, HID), lambda i: (0, i, 0)),
            pl.BlockSpec((P, BT, H), lambda i: (0, i, 0)),
            pl.BlockSpec((P, BT, H), lambda i: (0, i, 0)),
            pl.BlockSpec((BT, HID), lambda i: (i, 0)),
        ],
        out_shape=[
            jax.ShapeDtypeStruct((P, NPAD, HID), jnp.float32),
            jax.ShapeDtypeStruct((P, NPAD, H), jnp.float32),
            jax.ShapeDtypeStruct((P, NPAD, H), jnp.float32),
            jax.ShapeDtypeStruct((NPAD, HID), jnp.float32),
        ],
    )(x_pad, W_proj, b_proj, watt, asrc, adst, W_pat)


def _sc_exp_neg(x):
    y = x * 1.4426950408889634
    n = jnp.maximum((y - 0.5).astype(jnp.int32), -120)
    t = (y - n.astype(jnp.float32)) * 0.6931471805599453
    p = jnp.float32(1.0 / 720.0)
    for c in (1.0 / 120.0, 1.0 / 24.0, 1.0 / 6.0, 0.5, 1.0, 1.0):
        p = p * t + jnp.float32(c)
    s = plsc.bitcast((n + 127) << 23, jnp.float32)
    return p * s


def _sc_body(hp_hbm, s_hbm, ed_hbm, idx_hbm, z_hbm,
             s_tab, idx_v, hn_v, ed_v, z_v, sem_in, sem_hn, sem_out):
    lanes = lax.broadcasted_iota(jnp.int32, (16,), 0)
    wid = lax.axis_index("s") * 2 + lax.axis_index("c")
    base = pl.multiple_of(wid * CPW, B)

    @pl.loop(0, P)
    def _per_metapath(p):
        soff = pl.multiple_of(p * (NPAD * H), 8)
        pltpu.sync_copy(s_hbm.at[pl.ds(soff, NPAD * H)], s_tab)
        poff = p * NPAD

        def in_descs(b):
            slot = jnp.bitwise_and(b, 3)
            node0 = base + b * B
            ioff = pl.multiple_of((poff + node0) * K, 8)
            ib = pl.multiple_of(slot * (B * K), 8)
            edoff = pl.multiple_of((poff + node0) * H, 8)
            eb = pl.multiple_of(slot * (B * H), 8)
            di = pltpu.make_async_copy(
                idx_hbm.at[pl.ds(ioff, B * K)],
                idx_v.at[pl.ds(ib, B * K)], sem_in.at[slot])
            de = pltpu.make_async_copy(
                ed_hbm.at[pl.ds(edoff, B * H)],
                ed_v.at[pl.ds(eb, B * H)], sem_in.at[slot])
            return di, de

        def g_desc(b):
            slot = jnp.bitwise_and(b, 3)
            hslot = jnp.bitwise_and(b, 1)
            ib = pl.multiple_of(slot * (B * K), 8)
            hb = pl.multiple_of(hslot * (B * K), 8)
            return pltpu.make_async_copy(
                hp_hbm.at[idx_v.at[pl.ds(ib, B * K)]],
                hn_v.at[pl.ds(hb, B * K)], sem_hn.at[hslot])

        def out_desc(b):
            slot = jnp.bitwise_and(b, 3)
            node0 = base + b * B
            zoff = pl.multiple_of(poff + node0, 8)
            zb = pl.multiple_of(slot * B, 8)
            return pltpu.make_async_copy(
                z_v.at[pl.ds(zb, B)], z_hbm.at[pl.ds(zoff, B)],
                sem_out.at[slot])

        for d in in_descs(0) + in_descs(1):
            d.start()
        for d in in_descs(0):
            d.wait()
        g_desc(0).start()

        @pl.loop(0, NBLK)
        def _per_block(blk):
            slot = jnp.bitwise_and(blk, 3)
            hslot = jnp.bitwise_and(blk, 1)
            iboff = pl.multiple_of(slot * (B * K), 8)
            eboff = pl.multiple_of(slot * (B * H), 8)
            hboff = pl.multiple_of(hslot * (B * K), 8)
            zboff = pl.multiple_of(slot * B, 8)

            @pl.when(blk + 2 < NBLK)
            def _():
                for d in in_descs(blk + 2):
                    d.start()

            @pl.when(blk + 1 < NBLK)
            def _():
                for d in in_descs(blk + 1):
                    d.wait()
                g_desc(blk + 1).start()

            g_desc(blk).wait()

            @pl.when(blk >= 4)
            def _():
                out_desc(blk - 4).wait()

            for nb in range(B):
                iv_local = idx_v[pl.ds(iboff + nb * K, K)] - poff
                iv4 = iv_local * H
                us = []
                den16 = jnp.ones((16,), jnp.float32)
                for h in range(H):
                    es = plsc.load_gather(s_tab, [iv4 + h])
                    edb = plsc.load_gather(
                        ed_v, [jnp.full((K,), nb * H + h, jnp.int32) + eboff])
                    xv = edb + es
                    e = jnp.maximum(xv, 0.2 * xv)
                    ex = _sc_exp_neg(e - jnp.max(e))
                    us.append(ex)
                    den16 = jnp.where(lanes == h, jnp.sum(ex), den16)
                accs = [jnp.zeros((16,), jnp.float32) for _ in range(8)]
                for k in range(K):
                    r = nb * K + k
                    for h in range(H):
                        a = us[h][k]
                        accs[2 * h] += a * hn_v[hboff + r, pl.ds(h * DH, 16)]
                        accs[2 * h + 1] += a * hn_v[hboff + r,
                                                    pl.ds(h * DH + 16, 16)]
                for j in range(8):
                    z_v[zboff + nb, pl.ds(j * 16, 16)] = accs[j]
                z_v[zboff + nb, pl.ds(HID, 16)] = den16

            out_desc(blk).start()

        for b in range(NBLK - 4, NBLK):
            out_desc(b).wait()


def _stage2(hp_flat, s_flat, ed_flat, idx_flat):
    mesh = plsc.VectorSubcoreMesh(core_axis_name="c", subcore_axis_name="s")
    return pl.kernel(
        _sc_body,
        out_type=jax.ShapeDtypeStruct((P * NPAD, ZW), jnp.float32),
        mesh=mesh,
        compiler_params=pltpu.CompilerParams(needs_layout_passes=False),
        scratch_types=[
            pltpu.VMEM((NPAD * H,), jnp.float32),
            pltpu.VMEM((4 * B * K,), jnp.int32),
            pltpu.VMEM((2 * B * K, HID), jnp.float32),
            pltpu.VMEM((4 * B * H,), jnp.float32),
            pltpu.VMEM((4 * B, ZW), jnp.float32),
            pltpu.SemaphoreType.DMA((4,)),
            pltpu.SemaphoreType.DMA((2,)),
            pltpu.SemaphoreType.DMA((4,)),
        ],
    )(hp_flat, s_flat, ed_flat, idx_flat)


def _s3_body(z_ref, hpat_ref, wsem_ref, bsem_ref, q_ref, wout_ref,
             bout_ref, wclf_ref, bclf_ref, wreg_ref, breg_ref,
             logit_ref, score_ref, zf_ref, beta_ref):
    hp = hpat_ref[...]
    q = q_ref[...]
    row = lax.broadcasted_iota(jnp.int32, (16, HID), 0)
    col = lax.broadcasted_iota(jnp.int32, (16, HID), 1) // DH
    E = (row == col).astype(jnp.float32)
    zg, w = [], []
    for p in range(P):
        zrow = z_ref[p]
        den128 = _dot(zrow[:, HID:], E)
        zp = jax.nn.gelu(zrow[:, :HID] / den128)
        zg.append(zp)
        sf = jnp.tanh(_dot(zp, wsem_ref[...]) + hp + bsem_ref[...])
        w.append(jnp.sum(sf * q, axis=-1, keepdims=True))
    m = jnp.maximum(jnp.maximum(w[0], w[1]), w[2])
    e = [jnp.exp(wi - m) for wi in w]
    den = e[0] + e[1] + e[2]
    beta = [ei / den for ei in e]
    zf = beta[0] * zg[0] + beta[1] * zg[1] + beta[2] * zg[2]
    z = jax.nn.gelu(_dot(zf, wout_ref[...]) + bout_ref[...])
    logit_ref[...] = _dot(z, wclf_ref[...]) + bclf_ref[...]
    score_ref[...] = jax.nn.sigmoid(_dot(z, wreg_ref[...]) + breg_ref[...])
    zf_ref[...] = z
    beta_ref[...] = jnp.concatenate(beta, axis=1)


def _stage3(z_cat, hpat, W_sem, b_sem, q_sem, W_out, b_out, wclf2,
            bclf2, W_reg, b_reg):
    grid = (NPAD // BT,)
    return pl.pallas_call(
        _s3_body,
        grid=grid,
        in_specs=[
            pl.BlockSpec((P, BT, ZW), lambda i: (0, i, 0)),
            pl.BlockSpec((BT, HID), lambda i: (i, 0)),
            pl.BlockSpec((HID, HID), lambda i: (0, 0)),
            pl.BlockSpec((1, HID), lambda i: (0, 0)),
            pl.BlockSpec((1, HID), lambda i: (0, 0)),
            pl.BlockSpec((HID, OUT), lambda i: (0, 0)),
            pl.BlockSpec((1, OUT), lambda i: (0, 0)),
            pl.BlockSpec((OUT, ORG * SEV), lambda i: (0, 0)),
            pl.BlockSpec((1, ORG * SEV), lambda i: (0, 0)),
            pl.BlockSpec((OUT, ORG), lambda i: (0, 0)),
            pl.BlockSpec((1, ORG), lambda i: (0, 0)),
        ],
        out_specs=[
            pl.BlockSpec((BT, ORG * SEV), lambda i: (i, 0)),
            pl.BlockSpec((BT, ORG), lambda i: (i, 0)),
            pl.BlockSpec((BT, OUT), lambda i: (i, 0)),
            pl.BlockSpec((BT, P), lambda i: (i, 0)),
        ],
        out_shape=[
            jax.ShapeDtypeStruct((NPAD, ORG * SEV), jnp.float32),
            jax.ShapeDtypeStruct((NPAD, ORG), jnp.float32),
            jax.ShapeDtypeStruct((NPAD, OUT), jnp.float32),
            jax.ShapeDtypeStruct((NPAD, P), jnp.float32),
        ],
    )(z_cat, hpat, W_sem, b_sem, q_sem, W_out, b_out, wclf2, bclf2,
      W_reg, b_reg)


def kernel(patient_feats,
           neigh_idx_0, neigh_mask_0,
           neigh_idx_1, neigh_mask_1,
           neigh_idx_2, neigh_mask_2,
           W_proj, b_proj,
           W_att_0, a_src_0, a_dst_0,
           W_att_1, a_src_1, a_dst_1,
           W_att_2, a_src_2, a_dst_2,
           W_sem, W_pat, b_sem, q_sem,
           W_out, b_out, W_clf, b_clf, W_reg, b_reg):
    x_pad = jnp.pad(patient_feats, ((0, NPAD - N), (0, 0)))
    idx_cat = jnp.stack([
        jnp.pad(neigh_idx_0.astype(jnp.int32), ((0, NPAD - N), (0, 0))),
        jnp.pad(neigh_idx_1.astype(jnp.int32), ((0, NPAD - N), (0, 0)))
        + NPAD,
        jnp.pad(neigh_idx_2.astype(jnp.int32), ((0, NPAD - N), (0, 0)))
        + 2 * NPAD,
    ])
    watt = jnp.stack([W_att_0, W_att_1, W_att_2])
    asrc = jnp.stack([a_src_0.reshape(-1), a_src_1.reshape(-1),
                      a_src_2.reshape(-1)])
    adst = jnp.stack([a_dst_0.reshape(-1), a_dst_1.reshape(-1),
                      a_dst_2.reshape(-1)])

    hp_cat, s_cat, ed_cat, hpat = _stage1(
        x_pad, W_proj, b_proj.reshape(1, HID), watt, asrc, adst, W_pat)

    z_cat = _stage2(hp_cat.reshape(P * NPAD, HID), s_cat.reshape(-1),
                    ed_cat.reshape(-1), idx_cat.reshape(-1))

    wclf2 = jnp.transpose(W_clf, (1, 0, 2)).reshape(OUT, ORG * SEV)
    logits, scores, zf, beta = _stage3(
        z_cat.reshape(P, NPAD, ZW), hpat,
        W_sem, b_sem.reshape(1, HID),
        q_sem.reshape(1, HID), W_out, b_out.reshape(1, OUT), wclf2,
        b_clf.reshape(1, ORG * SEV), W_reg, b_reg.reshape(1, ORG))

    return (logits[:N].reshape(N, ORG, SEV), scores[:N], zf[:N], beta[:N])


# speedup vs baseline: 1.0017x; 1.0017x over previous
import functools

import jax
import jax.numpy as jnp
from jax import lax
from jax.experimental import pallas as pl
from jax.experimental.pallas import tpu as pltpu
from jax.experimental.pallas import tpu_sc as plsc

N = 10000
IN = 128
HID = 128
OUT = 128
H = 4
DH = 32
K = 16
P = 3
ORG = 25
SEV = 4

NPAD = 10240
NW = 32
CPW = NPAD // NW
B = 8
NBLK = CPW // B
ZW = HID + 16
BT = 512

_dot = functools.partial(jnp.dot, precision=lax.Precision.HIGHEST,
                         preferred_element_type=jnp.float32)


def _s1_body(x_ref, wp_ref, bp_ref, watt_ref, asrc_ref, adst_ref, wpat_ref,
             hp_ref, s_ref, ed_ref, hpat_ref):
    x = x_ref[...]
    h = jax.nn.gelu(_dot(x, wp_ref[...]) + bp_ref[...])
    hpat_ref[...] = _dot(h, wpat_ref[...])
    row = lax.broadcasted_iota(jnp.int32, (HID, H), 0) // DH
    col = lax.broadcasted_iota(jnp.int32, (HID, H), 1)
    M = (row == col).astype(jnp.float32)
    for p in range(P):
        hp = _dot(h, watt_ref[p])
        hp_ref[p] = hp
        s_ref[p] = _dot(hp * asrc_ref[p][None, :], M)
        ed_ref[p] = _dot(hp * adst_ref[p][None, :], M)


def _stage1(x_pad, W_proj, b_proj, watt, asrc, adst, W_pat):
    grid = (NPAD // BT,)
    return pl.pallas_call(
        _s1_body,
        grid=grid,
        in_specs=[
            pl.BlockSpec((BT, IN), lambda i: (i, 0)),
            pl.BlockSpec((IN, HID), lambda i: (0, 0)),
            pl.BlockSpec((1, HID), lambda i: (0, 0)),
            pl.BlockSpec((P, HID, HID), lambda i: (0, 0, 0)),
            pl.BlockSpec((P, HID), lambda i: (0, 0)),
            pl.BlockSpec((P, HID), lambda i: (0, 0)),
            pl.BlockSpec((HID, HID), lambda i: (0, 0)),
        ],
        out_specs=[
            pl.BlockSpec((P, BT, HID), lambda i: (0, i, 0)),
            pl.BlockSpec((P, BT, H), lambda i: (0, i, 0)),
            pl.BlockSpec((P, BT, H), lambda i: (0, i, 0)),
            pl.BlockSpec((BT, HID), lambda i: (i, 0)),
        ],
        out_shape=[
            jax.ShapeDtypeStruct((P, NPAD, HID), jnp.float32),
            jax.ShapeDtypeStruct((P, NPAD, H), jnp.float32),
            jax.ShapeDtypeStruct((P, NPAD, H), jnp.float32),
            jax.ShapeDtypeStruct((NPAD, HID), jnp.float32),
        ],
    )(x_pad, W_proj, b_proj, watt, asrc, adst, W_pat)


def _sc_exp_neg(x):
    y = x * 1.4426950408889634
    n = jnp.maximum((y - 0.5).astype(jnp.int32), -120)
    t = (y - n.astype(jnp.float32)) * 0.6931471805599453
    p = jnp.float32(1.0 / 720.0)
    for c in (1.0 / 120.0, 1.0 / 24.0, 1.0 / 6.0, 0.5, 1.0, 1.0):
        p = p * t + jnp.float32(c)
    s = plsc.bitcast((n + 127) << 23, jnp.float32)
    return p * s


def _sc_body(hp_hbm, s_hbm, ed_hbm, idx_hbm, z_hbm,
             s_tab, idx_v, hn_v, ed_v, z_v, sem_in, sem_hn, sem_out):
    lanes = lax.broadcasted_iota(jnp.int32, (16,), 0)
    wid = lax.axis_index("s") * 2 + lax.axis_index("c")
    base = pl.multiple_of(wid * CPW, B)

    @pl.loop(0, P)
    def _per_metapath(p):
        soff = pl.multiple_of(p * (NPAD * H), 8)
        pltpu.sync_copy(s_hbm.at[pl.ds(soff, NPAD * H)], s_tab)
        poff = p * NPAD

        def in_descs(b):
            slot = jnp.bitwise_and(b, 3)
            node0 = base + b * B
            ioff = pl.multiple_of((poff + node0) * K, 8)
            ib = pl.multiple_of(slot * (B * K), 8)
            edoff = pl.multiple_of((poff + node0) * H, 8)
            eb = pl.multiple_of(slot * (B * H), 8)
            di = pltpu.make_async_copy(
                idx_hbm.at[pl.ds(ioff, B * K)],
                idx_v.at[pl.ds(ib, B * K)], sem_in.at[slot])
            de = pltpu.make_async_copy(
                ed_hbm.at[pl.ds(edoff, B * H)],
                ed_v.at[pl.ds(eb, B * H)], sem_in.at[slot])
            return di, de

        def g_desc(b):
            slot = jnp.bitwise_and(b, 3)
            hslot = jnp.bitwise_and(b, 1)
            ib = pl.multiple_of(slot * (B * K), 8)
            hb = pl.multiple_of(hslot * (B * K), 8)
            return pltpu.make_async_copy(
                hp_hbm.at[idx_v.at[pl.ds(ib, B * K)]],
                hn_v.at[pl.ds(hb, B * K)], sem_hn.at[hslot])

        def out_desc(b):
            slot = jnp.bitwise_and(b, 3)
            node0 = base + b * B
            zoff = pl.multiple_of(poff + node0, 8)
            zb = pl.multiple_of(slot * B, 8)
            return pltpu.make_async_copy(
                z_v.at[pl.ds(zb, B)], z_hbm.at[pl.ds(zoff, B)],
                sem_out.at[slot])

        for d in in_descs(0) + in_descs(1):
            d.start()
        for d in in_descs(0):
            d.wait()
        g_desc(0).start()

        @pl.loop(0, NBLK)
        def _per_block(blk):
            slot = jnp.bitwise_and(blk, 3)
            hslot = jnp.bitwise_and(blk, 1)
            iboff = pl.multiple_of(slot * (B * K), 8)
            eboff = pl.multiple_of(slot * (B * H), 8)
            hboff = pl.multiple_of(hslot * (B * K), 8)
            zboff = pl.multiple_of(slot * B, 8)

            @pl.when(blk + 2 < NBLK)
            def _():
                for d in in_descs(blk + 2):
                    d.start()

            @pl.when(blk + 1 < NBLK)
            def _():
                for d in in_descs(blk + 1):
                    d.wait()
                g_desc(blk + 1).start()

            g_desc(blk).wait()

            @pl.when(blk >= 4)
            def _():
                out_desc(blk - 4).wait()

            for nb in range(B):
                iv_local = idx_v[pl.ds(iboff + nb * K, K)] - poff
                iv4 = iv_local * H
                us = []
                den16 = jnp.ones((16,), jnp.float32)
                for h in range(H):
                    es = plsc.load_gather(s_tab, [iv4 + h])
                    edb = plsc.load_gather(
                        ed_v, [jnp.full((K,), nb * H + h, jnp.int32) + eboff])
                    xv = edb + es
                    e = jnp.maximum(xv, 0.2 * xv)
                    ex = _sc_exp_neg(e - jnp.max(e))
                    us.append(ex)
                    den16 = jnp.where(lanes == h, jnp.sum(ex), den16)
                accs = [jnp.zeros((16,), jnp.float32) for _ in range(8)]
                for k in range(K):
                    r = nb * K + k
                    for h in range(H):
                        a = us[h][k]
                        accs[2 * h] += a * hn_v[hboff + r, pl.ds(h * DH, 16)]
                        accs[2 * h + 1] += a * hn_v[hboff + r,
                                                    pl.ds(h * DH + 16, 16)]
                for j in range(8):
                    z_v[zboff + nb, pl.ds(j * 16, 16)] = accs[j]
                z_v[zboff + nb, pl.ds(HID, 16)] = den16

            out_desc(blk).start()

        for b in range(NBLK - 4, NBLK):
            out_desc(b).wait()


def _stage2(hp_flat, s_flat, ed_flat, idx_flat):
    mesh = plsc.VectorSubcoreMesh(core_axis_name="c", subcore_axis_name="s")
    return pl.kernel(
        _sc_body,
        out_type=jax.ShapeDtypeStruct((P * NPAD, ZW), jnp.float32),
        mesh=mesh,
        compiler_params=pltpu.CompilerParams(needs_layout_passes=False),
        scratch_types=[
            pltpu.VMEM((NPAD * H,), jnp.float32),
            pltpu.VMEM((4 * B * K,), jnp.int32),
            pltpu.VMEM((2 * B * K, HID), jnp.float32),
            pltpu.VMEM((4 * B * H,), jnp.float32),
            pltpu.VMEM((4 * B, ZW), jnp.float32),
            pltpu.SemaphoreType.DMA((4,)),
            pltpu.SemaphoreType.DMA((2,)),
            pltpu.SemaphoreType.DMA((4,)),
        ],
    )(hp_flat, s_flat, ed_flat, idx_flat)


def _s3_body(z_ref, hpat_ref, wsem_ref, bsem_ref, q_ref, wout_ref,
             bout_ref, wclf_ref, bclf_ref, wreg_ref, breg_ref,
             logit_ref, score_ref, zf_ref, beta_ref):
    hp = hpat_ref[...]
    q = q_ref[...]
    row = lax.broadcasted_iota(jnp.int32, (16, HID), 0)
    col = lax.broadcasted_iota(jnp.int32, (16, HID), 1) // DH
    E = (row == col).astype(jnp.float32)
    zg, w = [], []
    for p in range(P):
        zrow = z_ref[p]
        den128 = _dot(zrow[:, HID:], E)
        zp = jax.nn.gelu(zrow[:, :HID] / den128)
        zg.append(zp)
        sf = jnp.tanh(_dot(zp, wsem_ref[...]) + hp + bsem_ref[...])
        w.append(jnp.sum(sf * q, axis=-1, keepdims=True))
    m = jnp.maximum(jnp.maximum(w[0], w[1]), w[2])
    e = [jnp.exp(wi - m) for wi in w]
    den = e[0] + e[1] + e[2]
    beta = [ei / den for ei in e]
    zf = beta[0] * zg[0] + beta[1] * zg[1] + beta[2] * zg[2]
    z = jax.nn.gelu(_dot(zf, wout_ref[...]) + bout_ref[...])
    logit_ref[...] = _dot(z, wclf_ref[...]) + bclf_ref[...]
    score_ref[...] = jax.nn.sigmoid(_dot(z, wreg_ref[...]) + breg_ref[...])
    zf_ref[...] = z
    beta_ref[...] = jnp.concatenate(beta, axis=1)


def _stage3(z_cat, hpat, W_sem, b_sem, q_sem, W_out, b_out, wclf2,
            bclf2, W_reg, b_reg):
    grid = (NPAD // BT,)
    return pl.pallas_call(
        _s3_body,
        grid=grid,
        in_specs=[
            pl.BlockSpec((P, BT, ZW), lambda i: (0, i, 0)),
            pl.BlockSpec((BT, HID), lambda i: (i, 0)),
            pl.BlockSpec((HID, HID), lambda i: (0, 0)),
            pl.BlockSpec((1, HID), lambda i: (0, 0)),
            pl.BlockSpec((1, HID), lambda i: (0, 0)),
            pl.BlockSpec((HID, OUT), lambda i: (0, 0)),
            pl.BlockSpec((1, OUT), lambda i: (0, 0)),
            pl.BlockSpec((OUT, ORG * SEV), lambda i: (0, 0)),
            pl.BlockSpec((1, ORG * SEV), lambda i: (0, 0)),
            pl.BlockSpec((OUT, ORG), lambda i: (0, 0)),
            pl.BlockSpec((1, ORG), lambda i: (0, 0)),
        ],
        out_specs=[
            pl.BlockSpec((BT, ORG * SEV), lambda i: (i, 0)),
            pl.BlockSpec((BT, ORG), lambda i: (i, 0)),
            pl.BlockSpec((BT, OUT), lambda i: (i, 0)),
            pl.BlockSpec((BT, P), lambda i: (i, 0)),
        ],
        out_shape=[
            jax.ShapeDtypeStruct((NPAD, ORG * SEV), jnp.float32),
            jax.ShapeDtypeStruct((NPAD, ORG), jnp.float32),
            jax.ShapeDtypeStruct((NPAD, OUT), jnp.float32),
            jax.ShapeDtypeStruct((NPAD, P), jnp.float32),
        ],
    )(z_cat, hpat, W_sem, b_sem, q_sem, W_out, b_out, wclf2, bclf2,
      W_reg, b_reg)


def kernel(patient_feats,
           neigh_idx_0, neigh_mask_0,
           neigh_idx_1, neigh_mask_1,
           neigh_idx_2, neigh_mask_2,
           W_proj, b_proj,
           W_att_0, a_src_0, a_dst_0,
           W_att_1, a_src_1, a_dst_1,
           W_att_2, a_src_2, a_dst_2,
           W_sem, W_pat, b_sem, q_sem,
           W_out, b_out, W_clf, b_clf, W_reg, b_reg):
    x_pad = jnp.pad(patient_feats, ((0, NPAD - N), (0, 0)))
    idx_cat = jnp.stack([
        jnp.pad(neigh_idx_0.astype(jnp.int32), ((0, NPAD - N), (0, 0))),
        jnp.pad(neigh_idx_1.astype(jnp.int32), ((0, NPAD - N), (0, 0)))
        + NPAD,
        jnp.pad(neigh_idx_2.astype(jnp.int32), ((0, NPAD - N), (0, 0)))
        + 2 * NPAD,
    ])
    watt = jnp.stack([W_att_0, W_att_1, W_att_2])
    asrc = jnp.stack([a_src_0.reshape(-1), a_src_1.reshape(-1),
                      a_src_2.reshape(-1)])
    adst = jnp.stack([a_dst_0.reshape(-1), a_dst_1.reshape(-1),
                      a_dst_2.reshape(-1)])

    hp_cat, s_cat, ed_cat, hpat = _stage1(
        x_pad, W_proj, b_proj.reshape(1, HID), watt, asrc, adst, W_pat)

    z_cat = _stage2(hp_cat.reshape(P * NPAD, HID), s_cat.reshape(-1),
                    ed_cat.reshape(-1), idx_cat.reshape(-1))

    wclf2 = jnp.transpose(W_clf, (1, 0, 2)).reshape(OUT, ORG * SEV)
    logits, scores, zf, beta = _stage3(
        z_cat.reshape(P, NPAD, ZW), hpat,
        W_sem, b_sem.reshape(1, HID),
        q_sem.reshape(1, HID), W_out, b_out.reshape(1, OUT), wclf2,
        b_clf.reshape(1, ORG * SEV), W_reg, b_reg.reshape(1, ORG))

    return (logits[:N].reshape(N, ORG, SEV), scores[:N], zf[:N], beta[:N])


# speedup vs baseline: 1.1075x; 1.1056x over previous
import functools

import jax
import jax.numpy as jnp
from jax import lax
from jax.experimental import pallas as pl
from jax.experimental.pallas import tpu as pltpu
from jax.experimental.pallas import tpu_sc as plsc

N = 10000
IN = 128
HID = 128
OUT = 128
H = 4
DH = 32
K = 16
P = 3
ORG = 25
SEV = 4

NPAD = 10240
NW = 32
CPW = NPAD // NW
B = 8
NBLK = CPW // B
ZW = HID + 16
BT = 512

_dot = functools.partial(jnp.dot, precision=lax.Precision.HIGHEST,
                         preferred_element_type=jnp.float32)


def _s1_body(x_ref, wp_ref, bp_ref, watt_ref, asrc_ref, adst_ref, wpat_ref,
             hp_ref, s_ref, ed_ref, hpat_ref):
    x = x_ref[...]
    h = jax.nn.gelu(_dot(x, wp_ref[...]) + bp_ref[...])
    hpat_ref[...] = _dot(h, wpat_ref[...])
    row = lax.broadcasted_iota(jnp.int32, (HID, H), 0) // DH
    col = lax.broadcasted_iota(jnp.int32, (HID, H), 1)
    M = (row == col).astype(jnp.float32)
    for p in range(P):
        hp = _dot(h, watt_ref[p])
        hp_ref[p] = hp
        s_ref[p] = _dot(hp * asrc_ref[p][None, :], M)
        ed_ref[p] = _dot(hp * adst_ref[p][None, :], M)


def _stage1(x_pad, W_proj, b_proj, watt, asrc, adst, W_pat):
    grid = (NPAD // BT,)
    return pl.pallas_call(
        _s1_body,
        grid=grid,
        in_specs=[
            pl.BlockSpec((BT, IN), lambda i: (i, 0)),
            pl.BlockSpec((IN, HID), lambda i: (0, 0)),
            pl.BlockSpec((1, HID), lambda i: (0, 0)),
            pl.BlockSpec((P, HID, HID), lambda i: (0, 0, 0)),
            pl.BlockSpec((P, HID), lambda i: (0, 0)),
            pl.BlockSpec((P, HID), lambda i: (0, 0)),
            pl.BlockSpec((HID, HID), lambda i: (0, 0)),
        ],
        out_specs=[
            pl.BlockSpec((P, BT, HID), lambda i: (0, i, 0)),
            pl.BlockSpec((P, BT, H), lambda i: (0, i, 0)),
            pl.BlockSpec((P, BT, H), lambda i: (0, i, 0)),
            pl.BlockSpec((BT, HID), lambda i: (i, 0)),
        ],
        out_shape=[
            jax.ShapeDtypeStruct((P, NPAD, HID), jnp.float32),
            jax.ShapeDtypeStruct((P, NPAD, H), jnp.float32),
            jax.ShapeDtypeStruct((P, NPAD, H), jnp.float32),
            jax.ShapeDtypeStruct((NPAD, HID), jnp.float32),
        ],
    )(x_pad, W_proj, b_proj, watt, asrc, adst, W_pat)


def _sc_exp_neg(x):
    y = x * 1.4426950408889634
    n = jnp.maximum((y - 0.5).astype(jnp.int32), -120)
    t = (y - n.astype(jnp.float32)) * 0.6931471805599453
    p = jnp.float32(1.0 / 720.0)
    for c in (1.0 / 120.0, 1.0 / 24.0, 1.0 / 6.0, 0.5, 1.0, 1.0):
        p = p * t + jnp.float32(c)
    s = plsc.bitcast((n + 127) << 23, jnp.float32)
    return p * s


def _sc_body(hp_hbm, s_hbm, ed_hbm, idx_hbm, z_hbm, den_hbm,
             s_tab, idx_v, hn_v, ed_v, z_v, den_v, sem_in, sem_hn, sem_out):
    lanes = lax.broadcasted_iota(jnp.int32, (16,), 0)
    wid = lax.axis_index("s") * 2 + lax.axis_index("c")
    base = pl.multiple_of(wid * CPW, B)

    @pl.loop(0, P)
    def _per_metapath(p):
        soff = pl.multiple_of(p * (NPAD * H), 8)
        pltpu.sync_copy(s_hbm.at[pl.ds(soff, NPAD * H)], s_tab)
        poff = p * NPAD

        def in_descs(b):
            slot = jnp.bitwise_and(b, 3)
            node0 = base + b * B
            ioff = pl.multiple_of((poff + node0) * K, 8)
            ib = pl.multiple_of(slot * (B * K), 8)
            edoff = pl.multiple_of((poff + node0) * H, 8)
            eb = pl.multiple_of(slot * (B * H), 8)
            di = pltpu.make_async_copy(
                idx_hbm.at[pl.ds(ioff, B * K)],
                idx_v.at[pl.ds(ib, B * K)], sem_in.at[slot])
            de = pltpu.make_async_copy(
                ed_hbm.at[pl.ds(edoff, B * H)],
                ed_v.at[pl.ds(eb, B * H)], sem_in.at[slot])
            return di, de

        def g_desc(b):
            slot = jnp.bitwise_and(b, 3)
            ib = pl.multiple_of(slot * (B * K), 8)
            hb = pl.multiple_of(slot * (B * K), 8)
            return pltpu.make_async_copy(
                hp_hbm.at[idx_v.at[pl.ds(ib, B * K)]],
                hn_v.at[pl.ds(hb, B * K)], sem_hn.at[slot])

        def out_descs(b):
            slot = jnp.bitwise_and(b, 3)
            node0 = base + b * B
            zoff = pl.multiple_of(poff + node0, 8)
            zb = pl.multiple_of(slot * B, 8)
            doff = pl.multiple_of((poff + node0) * 16, 8)
            db = pl.multiple_of(slot * (B * 16), 8)
            dz = pltpu.make_async_copy(
                z_v.at[pl.ds(zb, B)], z_hbm.at[pl.ds(zoff, B)],
                sem_out.at[slot])
            dd = pltpu.make_async_copy(
                den_v.at[pl.ds(db, B * 16)], den_hbm.at[pl.ds(doff, B * 16)],
                sem_out.at[slot])
            return dz, dd

        for d in in_descs(0) + in_descs(1):
            d.start()
        for d in in_descs(0):
            d.wait()
        g_desc(0).start()

        @pl.loop(0, NBLK)
        def _per_block(blk):
            slot = jnp.bitwise_and(blk, 3)
            iboff = pl.multiple_of(slot * (B * K), 8)
            eboff = pl.multiple_of(slot * (B * H), 8)
            hboff = pl.multiple_of(slot * (B * K), 8)
            zboff = pl.multiple_of(slot * B, 8)
            dboff = pl.multiple_of(slot * (B * 16), 8)

            @pl.when(blk + 2 < NBLK)
            def _():
                for d in in_descs(blk + 2):
                    d.start()

            @pl.when(blk + 1 < NBLK)
            def _():
                for d in in_descs(blk + 1):
                    d.wait()
                g_desc(blk + 1).start()

            g_desc(blk).wait()

            @pl.when(blk >= 4)
            def _():
                for d in out_descs(blk - 4):
                    d.wait()

            for nb in range(B):
                iv_local = idx_v[pl.ds(iboff + nb * K, K)] - poff
                iv4 = iv_local * H
                us = []
                den16 = jnp.ones((16,), jnp.float32)
                for h in range(H):
                    es = plsc.load_gather(s_tab, [iv4 + h])
                    edb = plsc.load_gather(
                        ed_v, [jnp.full((K,), nb * H + h, jnp.int32) + eboff])
                    xv = edb + es
                    e = jnp.maximum(xv, 0.2 * xv)
                    ex = _sc_exp_neg(e - jnp.max(e))
                    us.append(ex)
                    den16 = jnp.where(lanes == h, jnp.sum(ex), den16)
                accs = [jnp.zeros((16,), jnp.float32) for _ in range(8)]
                for k in range(K):
                    r = nb * K + k
                    for h in range(H):
                        a = us[h][k]
                        accs[2 * h] += a * hn_v[hboff + r, pl.ds(h * DH, 16)]
                        accs[2 * h + 1] += a * hn_v[hboff + r,
                                                    pl.ds(h * DH + 16, 16)]
                den_v[pl.ds(dboff + nb * 16, 16)] = den16
                for j in range(8):
                    z_v[zboff + nb, pl.ds(j * 16, 16)] = accs[j]

            for d in out_descs(blk):
                d.start()

        for b in range(NBLK - 4, NBLK):
            for d in out_descs(b):
                d.wait()


def _stage2(hp_flat, s_flat, ed_flat, idx_flat):
    mesh = plsc.VectorSubcoreMesh(core_axis_name="c", subcore_axis_name="s")
    return pl.kernel(
        _sc_body,
        out_type=[
            jax.ShapeDtypeStruct((P * NPAD, HID), jnp.float32),
            jax.ShapeDtypeStruct((P * NPAD * 16,), jnp.float32),
        ],
        mesh=mesh,
        compiler_params=pltpu.CompilerParams(needs_layout_passes=False),
        scratch_types=[
            pltpu.VMEM((NPAD * H,), jnp.float32),
            pltpu.VMEM((4 * B * K,), jnp.int32),
            pltpu.VMEM((4 * B * K, HID), jnp.float32),
            pltpu.VMEM((4 * B * H,), jnp.float32),
            pltpu.VMEM((4 * B, HID), jnp.float32),
            pltpu.VMEM((4 * B * 16,), jnp.float32),
            pltpu.SemaphoreType.DMA((4,)),
            pltpu.SemaphoreType.DMA((4,)),
            pltpu.SemaphoreType.DMA((4,)),
        ],
    )(hp_flat, s_flat, ed_flat, idx_flat)


def _s3_body(z_ref, den_ref, hpat_ref, wsem_ref, bsem_ref, q_ref, wout_ref,
             bout_ref, wclf_ref, bclf_ref, wreg_ref, breg_ref,
             logit_ref, score_ref, zf_ref, beta_ref):
    hp = hpat_ref[...]
    q = q_ref[...]
    row = lax.broadcasted_iota(jnp.int32, (16, HID), 0)
    col = lax.broadcasted_iota(jnp.int32, (16, HID), 1) // DH
    E = (row == col).astype(jnp.float32)
    zg, w = [], []
    for p in range(P):
        den128 = _dot(den_ref[p], E)
        zp = jax.nn.gelu(z_ref[p] / den128)
        zg.append(zp)
        sf = jnp.tanh(_dot(zp, wsem_ref[...]) + hp + bsem_ref[...])
        w.append(jnp.sum(sf * q, axis=-1, keepdims=True))
    m = jnp.maximum(jnp.maximum(w[0], w[1]), w[2])
    e = [jnp.exp(wi - m) for wi in w]
    den = e[0] + e[1] + e[2]
    beta = [ei / den for ei in e]
    zf = beta[0] * zg[0] + beta[1] * zg[1] + beta[2] * zg[2]
    z = jax.nn.gelu(_dot(zf, wout_ref[...]) + bout_ref[...])
    logit_ref[...] = _dot(z, wclf_ref[...]) + bclf_ref[...]
    score_ref[...] = jax.nn.sigmoid(_dot(z, wreg_ref[...]) + breg_ref[...])
    zf_ref[...] = z
    beta_ref[...] = jnp.concatenate(beta, axis=1)


def _stage3(z_cat, den_cat, hpat, W_sem, b_sem, q_sem, W_out, b_out, wclf2,
            bclf2, W_reg, b_reg):
    grid = (NPAD // BT,)
    return pl.pallas_call(
        _s3_body,
        grid=grid,
        in_specs=[
            pl.BlockSpec((P, BT, HID), lambda i: (0, i, 0)),
            pl.BlockSpec((P, BT, 16), lambda i: (0, i, 0)),
            pl.BlockSpec((BT, HID), lambda i: (i, 0)),
            pl.BlockSpec((HID, HID), lambda i: (0, 0)),
            pl.BlockSpec((1, HID), lambda i: (0, 0)),
            pl.BlockSpec((1, HID), lambda i: (0, 0)),
            pl.BlockSpec((HID, OUT), lambda i: (0, 0)),
            pl.BlockSpec((1, OUT), lambda i: (0, 0)),
            pl.BlockSpec((OUT, ORG * SEV), lambda i: (0, 0)),
            pl.BlockSpec((1, ORG * SEV), lambda i: (0, 0)),
            pl.BlockSpec((OUT, ORG), lambda i: (0, 0)),
            pl.BlockSpec((1, ORG), lambda i: (0, 0)),
        ],
        out_specs=[
            pl.BlockSpec((BT, ORG * SEV), lambda i: (i, 0)),
            pl.BlockSpec((BT, ORG), lambda i: (i, 0)),
            pl.BlockSpec((BT, OUT), lambda i: (i, 0)),
            pl.BlockSpec((BT, P), lambda i: (i, 0)),
        ],
        out_shape=[
            jax.ShapeDtypeStruct((NPAD, ORG * SEV), jnp.float32),
            jax.ShapeDtypeStruct((NPAD, ORG), jnp.float32),
            jax.ShapeDtypeStruct((NPAD, OUT), jnp.float32),
            jax.ShapeDtypeStruct((NPAD, P), jnp.float32),
        ],
    )(z_cat, den_cat, hpat, W_sem, b_sem, q_sem, W_out, b_out, wclf2, bclf2,
      W_reg, b_reg)


def kernel(patient_feats,
           neigh_idx_0, neigh_mask_0,
           neigh_idx_1, neigh_mask_1,
           neigh_idx_2, neigh_mask_2,
           W_proj, b_proj,
           W_att_0, a_src_0, a_dst_0,
           W_att_1, a_src_1, a_dst_1,
           W_att_2, a_src_2, a_dst_2,
           W_sem, W_pat, b_sem, q_sem,
           W_out, b_out, W_clf, b_clf, W_reg, b_reg):
    x_pad = jnp.pad(patient_feats, ((0, NPAD - N), (0, 0)))
    idx_cat = jnp.stack([
        jnp.pad(neigh_idx_0.astype(jnp.int32), ((0, NPAD - N), (0, 0))),
        jnp.pad(neigh_idx_1.astype(jnp.int32), ((0, NPAD - N), (0, 0)))
        + NPAD,
        jnp.pad(neigh_idx_2.astype(jnp.int32), ((0, NPAD - N), (0, 0)))
        + 2 * NPAD,
    ])
    watt = jnp.stack([W_att_0, W_att_1, W_att_2])
    asrc = jnp.stack([a_src_0.reshape(-1), a_src_1.reshape(-1),
                      a_src_2.reshape(-1)])
    adst = jnp.stack([a_dst_0.reshape(-1), a_dst_1.reshape(-1),
                      a_dst_2.reshape(-1)])

    hp_cat, s_cat, ed_cat, hpat = _stage1(
        x_pad, W_proj, b_proj.reshape(1, HID), watt, asrc, adst, W_pat)

    z_cat, den_cat = _stage2(hp_cat.reshape(P * NPAD, HID), s_cat.reshape(-1),
                             ed_cat.reshape(-1), idx_cat.reshape(-1))

    wclf2 = jnp.transpose(W_clf, (1, 0, 2)).reshape(OUT, ORG * SEV)
    logits, scores, zf, beta = _stage3(
        z_cat.reshape(P, NPAD, HID), den_cat.reshape(P, NPAD, 16), hpat,
        W_sem, b_sem.reshape(1, HID),
        q_sem.reshape(1, HID), W_out, b_out.reshape(1, OUT), wclf2,
        b_clf.reshape(1, ORG * SEV), W_reg, b_reg.reshape(1, ORG))

    return (logits[:N].reshape(N, ORG, SEV), scores[:N], zf[:N], beta[:N])
